# baseline (speedup 1.0000x reference)
import functools

import jax
import jax.numpy as jnp
from jax import lax
from jax.experimental import pallas as pl
from jax.experimental.pallas import tpu as pltpu
from jax.experimental.pallas import tpu_sc as plsc

_NI = 16384
_NJ = 200
_EMB = 32
_JB = _NJ // 8
_IB = _NI // 128
_JS = 4
_N_UNITS = _JB * _IB * 2
_NUM_WORKERS = 32
_IB_PER_W = _IB // _NUM_WORKERS
_SEG = _JB * 2
_PAIRS = _SEG // 2


def _make_sc_lookup():
    mesh = plsc.VectorSubcoreMesh(core_axis_name="c", subcore_axis_name="s")

    _PITCH = 515
    scratch = (
        [pltpu.VMEM((2, 128, _NJ), jnp.int32)]
        + [pltpu.VMEM((_JS * 128,), jnp.int32) for _ in range(2)]
        + [pltpu.VMEM((_JS * 128, _EMB), jnp.float32) for _ in range(2)]
        + [pltpu.VMEM((4, 8, _PITCH), jnp.float32) for _ in range(2)]
        + [pltpu.SemaphoreType.DMA for _ in range(5)]
    )

    @functools.partial(
        pl.kernel,
        mesh=mesh,
        out_type=jax.ShapeDtypeStruct((_NJ, 4, _IB, 8, 128), jnp.float32),
        scratch_types=scratch,
        compiler_params=pltpu.CompilerParams(
            use_tc_tiling_on_sc=False, needs_layout_passes=False
        ),
    )
    def emb_kernel(idx_hbm, table_hbm, out_hbm, *scr):
        idx_t = scr[0]
        idx_v = scr[1:3]
        rows_v = scr[3:5]
        stage_v = scr[5:7]
        st = scr[7]
        sg = scr[8:10]
        so = scr[10:12]

        wid = lax.axis_index("s") * 2 + lax.axis_index("c")
        ib0 = wid * _IB_PER_W
        iota16 = lax.iota(jnp.int32, 16)

        def j_start(v):
            jb = lax.div(v, 2)
            jh = lax.rem(v, 2)
            return jb * 8 + jh * _JS

        def start_tile(ib, t):
            pltpu.async_copy(
                idx_hbm.at[pl.ds(ib * 128, 128), :], idx_t.at[t], st
            )

        def wait_tile():
            pltpu.make_async_copy(
                idx_hbm.at[pl.ds(0, 128), :], idx_t.at[0], st
            ).wait()

        def repack_unit(v, t, b):
            j0 = j_start(v)

            t_vec = jnp.full((16,), t, jnp.int32)

            @plsc.parallel_loop(0, 8, unroll=1)
            def rbody(blk):
                row_vec = iota16 + blk * 16
                for ji in range(_JS):
                    col_vec = jnp.full((16,), j0 + ji, jnp.int32)
                    vals = plsc.load_gather(
                        idx_t, [t_vec, row_vec, col_vec]
                    )
                    idx_v[b][pl.ds(ji * 128 + blk * 16, 16)] = vals

        def start_gathers(b):
            pltpu.async_copy(table_hbm.at[idx_v[b]], rows_v[b], sg[b])

        def wait_gathers(b):
            pltpu.make_async_copy(
                table_hbm.at[idx_v[b]], rows_v[b], sg[b]
            ).wait()

        def start_out(v, ib, b):
            j0 = j_start(v)
            for ji in range(_JS):
                pltpu.async_copy(
                    stage_v[b].at[:, :, pl.ds(ji * 128, 128)],
                    out_hbm.at[j0 + ji, :, ib, :, :],
                    so[b],
                )

        def wait_out(b):
            for _ in range(_JS):
                pltpu.make_async_copy(
                    stage_v[b].at[:, :, pl.ds(0, 128)],
                    out_hbm.at[0, :, 0, :, :],
                    so[b],
                ).wait()

        def transpose_unit(b):
            db_vecs = [(iota16 + h * 16) // 8 for h in range(2)]
            di_vecs = [(iota16 + h * 16) % 8 for h in range(2)]

            @plsc.parallel_loop(0, 128, unroll=2)
            def tbody(ii):
                for ji in range(_JS):
                    r = ji * 128 + ii
                    col_vec = jnp.full((16,), r, jnp.int32)
                    for h in range(2):
                        v = rows_v[b][r, pl.ds(h * 16, 16)]
                        plsc.store_scatter(
                            stage_v[b], [db_vecs[h], di_vecs[h], col_vec], v
                        )

        def step(v, ib, t, b, repack2, launch_next, wait_prev_out):
            wait_gathers(b)
            if launch_next:
                start_gathers(1 - b)
            if repack2:
                repack_unit(v + 2, t, b)
            if wait_prev_out:
                wait_out(b)
            transpose_unit(b)
            start_out(v, ib, b)

        pltpu.sync_copy(idx_hbm.at[pl.ds(ib0 * 128, 128), :], idx_t.at[0])

        def segment(ib, t, first):
            repack_unit(0, t, 0)
            start_gathers(0)
            repack_unit(1, t, 1)
            step(0, ib, t, 0, repack2=True, launch_next=True,
                 wait_prev_out=not first)
            step(1, ib, t, 1, repack2=True, launch_next=True,
                 wait_prev_out=not first)

            def seg_body(g, _):
                v = g * 2
                step(v, ib, t, 0, repack2=True, launch_next=True,
                     wait_prev_out=True)
                step(v + 1, ib, t, 1, repack2=True, launch_next=True,
                     wait_prev_out=True)
                return ()

            lax.fori_loop(1, _PAIRS - 1, seg_body, (), unroll=False)

            v = (_PAIRS - 1) * 2
            step(v, ib, t, 0, repack2=False, launch_next=True,
                 wait_prev_out=True)
            step(v + 1, ib, t, 1, repack2=False, launch_next=False,
                 wait_prev_out=True)

        start_tile(ib0 + 1, 1)
        segment(ib0, 0, first=True)

        def kbody(k, _):
            t = lax.rem(k, 2)
            wait_tile()
            start_tile(ib0 + lax.rem(k + 1, _IB_PER_W), 1 - t)
            segment(ib0 + k, t, first=False)
            return ()

        lax.fori_loop(1, _IB_PER_W, kbody, (), unroll=False)

        wait_tile()
        wait_out(0)
        wait_out(1)

    return emb_kernel


def kernel(indices, table):
    out5 = _make_sc_lookup()(indices.astype(jnp.int32), table)
    return out5.transpose(2, 4, 0, 1, 3).reshape(_NI, _NJ, _EMB)
